# baseline (speedup 1.0000x reference)
import functools

import jax
import jax.numpy as jnp
from jax import lax
from jax.experimental import pallas as pl
from jax.experimental.pallas import tpu as pltpu
from jax.experimental.pallas import tpu_sc as plsc

_B = 16384
_D = 64
_NC = 2
_NS = 16
_NW = _NC * _NS
_B_PER_W = _B // _NW
_G = 16
_NG = _B_PER_W // _G


@functools.partial(
    pl.kernel,
    mesh=plsc.VectorSubcoreMesh(core_axis_name="c", subcore_axis_name="s"),
    out_type=jax.ShapeDtypeStruct((_B, _D), jnp.float32),
    scratch_types=[
        pltpu.VMEM((_B_PER_W,), jnp.int32),
        pltpu.VMEM((_B_PER_W // 2, _D), jnp.float32),
        pltpu.SemaphoreType.DMA,
        pltpu.SemaphoreType.DMA,
    ],
)
def _gather_kernel(x_hbm, table_hbm, out_hbm, idx_v, rows_v, sem, sem2):
    wid = lax.axis_index("s") * _NC + lax.axis_index("c")
    base = wid * _B_PER_W
    half = _B_PER_W // 2
    pltpu.sync_copy(x_hbm.at[pl.ds(base, _B_PER_W)], idx_v)

    def fire(g, _):
        vec = idx_v[pl.ds(g * _G, _G)]
        vec2 = idx_v[pl.ds(half + g * _G, _G)]
        for j in range(_G):
            pltpu.make_async_copy(
                table_hbm.at[pl.ds(vec[j], 1)],
                rows_v.at[pl.ds(g * _G + j, 1)],
                sem,
            ).start()
            pltpu.make_async_copy(
                table_hbm.at[pl.ds(vec2[j], 1)],
                out_hbm.at[pl.ds(base + half + g * _G + j, 1)],
                sem2,
            ).start()
        return _

    lax.fori_loop(0, _NG // 2, fire, 0)

    pltpu.make_async_copy(
        table_hbm.at[pl.ds(0, half)], rows_v, sem
    ).wait()
    pltpu.sync_copy(rows_v, out_hbm.at[pl.ds(base, half)])
    pltpu.make_async_copy(
        table_hbm.at[pl.ds(0, half)],
        out_hbm.at[pl.ds(base + half, half)],
        sem2,
    ).wait()


def kernel(x, table):
    return _gather_kernel(x.astype(jnp.int32), table)


# speedup vs baseline: 1.3232x; 1.3232x over previous
import functools

import jax
import jax.numpy as jnp
from jax import lax
from jax.experimental import pallas as pl
from jax.experimental.pallas import tpu as pltpu
from jax.experimental.pallas import tpu_sc as plsc

_B = 16384
_D = 64
_NC = 2
_NS = 16
_NW = _NC * _NS
_B_PER_W = _B // _NW
_G = 16
_NG = _B_PER_W // _G


@functools.partial(
    pl.kernel,
    mesh=plsc.VectorSubcoreMesh(core_axis_name="c", subcore_axis_name="s"),
    out_type=jax.ShapeDtypeStruct((_B, _D), jnp.float32),
    scratch_types=[
        pltpu.VMEM((_B_PER_W,), jnp.int32),
        pltpu.VMEM((_B_PER_W, _D), jnp.float32),
        pltpu.SemaphoreType.DMA,
        pltpu.SemaphoreType.DMA,
        pltpu.SemaphoreType.DMA,
        pltpu.SemaphoreType.DMA,
    ],
)
def _gather_kernel(x_hbm, table_hbm, out_hbm, idx_v, rows_v, s0, s1, s2, s3):
    wid = lax.axis_index("s") * _NC + lax.axis_index("c")
    base = wid * _B_PER_W
    pltpu.sync_copy(x_hbm.at[pl.ds(base, _B_PER_W)], idx_v)
    sems = [s0, s1, s2, s3]

    def fire(g, _):
        vec = idx_v[pl.ds(g * _G, _G)]
        for j in range(_G):
            row = vec[j]
            pltpu.make_async_copy(
                table_hbm.at[pl.ds(row, 1)],
                rows_v.at[pl.ds(g * _G + j, 1)],
                sems[j % 4],
            ).start()
        return _

    lax.fori_loop(0, _NG, fire, 0)

    for q in range(4):
        pltpu.make_async_copy(
            table_hbm.at[pl.ds(0, _B_PER_W // 4)],
            rows_v.at[pl.ds(q * (_B_PER_W // 4), _B_PER_W // 4)],
            sems[q],
        ).wait()
    pltpu.sync_copy(rows_v, out_hbm.at[pl.ds(base, _B_PER_W)])


def kernel(x, table):
    return _gather_kernel(x.astype(jnp.int32), table)
